# baseline (speedup 1.0000x reference)
import functools

import jax
import jax.numpy as jnp
from jax import lax
from jax.experimental import pallas as pl
from jax.experimental.pallas import tpu as pltpu
from jax.experimental.pallas import tpu_sc as plsc

NUM_HEADS = 16
NUM_SPATIAL = 512
N_DATA = 512
N_OUT = 513
BATCH = 4
L = 16
NC = 2
NS = 16
NW = NC * NS
ROWS_PER_W = 16
CROWS = 1
NCHUNK = ROWS_PER_W // CROWS
NRING = 8
VREGS_PER_ROW = N_DATA // L


def _sc_body(sp_hbm, tbl_hbm, out_hbm, tbl_v, sp_v, pad_v, buf_a, buf_b,
             buf_c, buf_d, buf_e, buf_f, buf_g, buf_h, sem, sem2):
    wid = lax.axis_index("s") * NC + lax.axis_index("c")
    d0 = pl.multiple_of(wid * ROWS_PER_W, 8)

    cp_tbl = pltpu.async_copy(tbl_hbm, tbl_v, sem)
    cp_sp = pltpu.async_copy(sp_hbm.at[pl.ds(d0, ROWS_PER_W), :], sp_v, sem)
    cp_tbl.wait()
    cp_sp.wait()

    iota = lax.iota(jnp.int32, L)
    zeros = jnp.zeros((L,), jnp.int32)
    pv = plsc.load_gather(tbl_v, [iota, zeros + NUM_SPATIAL])

    @plsc.parallel_loop(0, N_OUT, unroll=8)
    def _pr(j):
        plsc.store_scatter(pad_v, [zeros, iota, zeros + j], pv)

    @pl.when(wid < BATCH)
    def _send_pad_row():
        pltpu.async_copy(pad_v, out_hbm.at[wid, pl.ds(0, 1), :, :], sem2)

    def _gather_row(buf, r, spr):
        plsc.store_scatter(buf, [zeros + r, iota, zeros], pv)

        @plsc.parallel_loop(0, VREGS_PER_ROW, unroll=2)
        def _k(k):
            sv = plsc.load_gather(sp_v, [zeros + spr, iota + k * L])
            iv0 = (sv * jnp.float32(NUM_SPATIAL)).astype(jnp.int32)
            cvec = iota + (1 + k * L)
            for h in range(NUM_HEADS):
                vals = plsc.load_gather(tbl_v, [zeros + h, iv0])
                plsc.store_scatter(buf, [zeros + r, zeros + h, cvec], vals)

    bufs = (buf_a, buf_b, buf_c, buf_d, buf_e, buf_f, buf_g, buf_h)
    pending = {}
    for c in range(NCHUNK):
        buf = bufs[c % NRING]
        if c >= NRING:
            for cp in pending.pop(c - NRING):
                cp.wait()

        def _row(r, cc):
            _gather_row(buf, r, c * CROWS + r)
            return cc
        lax.fori_loop(0, CROWS, _row, 0)

        pending[c] = [
            pltpu.async_copy(
                buf,
                out_hbm.at[b, pl.ds(d0 + 1 + c * CROWS, CROWS), :, :], sem)
            for b in range(BATCH)
        ]
    for c in range(NCHUNK - NRING, NCHUNK):
        for cp in pending.pop(c):
            cp.wait()

    @pl.when(wid < BATCH)
    def _drain_pad_row():
        pltpu.make_async_copy(
            pad_v, out_hbm.at[wid, pl.ds(0, 1), :, :], sem2).wait()


@jax.jit
def _graph_attn_bias(spatial_pos, emb_weight):
    mesh = plsc.VectorSubcoreMesh(core_axis_name="c", subcore_axis_name="s")
    f = functools.partial(
        pl.kernel,
        mesh=mesh,
        out_type=jax.ShapeDtypeStruct((BATCH, N_OUT, NUM_HEADS, N_OUT),
                                      jnp.float32),
        scratch_types=[
            pltpu.VMEM((NUM_HEADS, NUM_SPATIAL + 1), jnp.float32),
            pltpu.VMEM((ROWS_PER_W, N_DATA), jnp.float32),
            pltpu.VMEM((1, NUM_HEADS, N_OUT), jnp.float32),
            pltpu.VMEM((CROWS, NUM_HEADS, N_OUT), jnp.float32),
            pltpu.VMEM((CROWS, NUM_HEADS, N_OUT), jnp.float32),
            pltpu.VMEM((CROWS, NUM_HEADS, N_OUT), jnp.float32),
            pltpu.VMEM((CROWS, NUM_HEADS, N_OUT), jnp.float32),
            pltpu.VMEM((CROWS, NUM_HEADS, N_OUT), jnp.float32),
            pltpu.VMEM((CROWS, NUM_HEADS, N_OUT), jnp.float32),
            pltpu.VMEM((CROWS, NUM_HEADS, N_OUT), jnp.float32),
            pltpu.VMEM((CROWS, NUM_HEADS, N_OUT), jnp.float32),
            pltpu.SemaphoreType.DMA,
            pltpu.SemaphoreType.DMA,
        ],
        compiler_params=pltpu.CompilerParams(needs_layout_passes=False),
    )(_sc_body)
    out_bihj = f(spatial_pos, emb_weight.T)
    return jnp.transpose(out_bihj, (0, 2, 1, 3))


def kernel(spatial_pos, x, emb_weight):
    del x
    return _graph_attn_bias(spatial_pos, emb_weight)


# speedup vs baseline: 1.0807x; 1.0807x over previous
import functools

import jax
import jax.numpy as jnp
from jax import lax
from jax.experimental import pallas as pl
from jax.experimental.pallas import tpu as pltpu
from jax.experimental.pallas import tpu_sc as plsc

NUM_HEADS = 16
NUM_SPATIAL = 512
N_DATA = 512
N_OUT = 513
BATCH = 4
L = 16
NC = 2
NS = 16
NW = NC * NS
ROWS_PER_W = 16
CROWS = 2
NCHUNK = ROWS_PER_W // CROWS
NRING = 4
VREGS_PER_ROW = N_DATA // L


def _sc_body(sp_hbm, tbl_hbm, out_hbm, tbl_v, sp_v, pad_v, buf_a, buf_b,
             buf_c, buf_d, sem, sem2):
    wid = lax.axis_index("s") * NC + lax.axis_index("c")
    d0 = pl.multiple_of(wid * ROWS_PER_W, 8)

    cp_tbl = pltpu.async_copy(tbl_hbm, tbl_v, sem)
    cp_sp = pltpu.async_copy(sp_hbm.at[pl.ds(d0, ROWS_PER_W), :], sp_v, sem)
    cp_tbl.wait()
    cp_sp.wait()

    iota = lax.iota(jnp.int32, L)
    zeros = jnp.zeros((L,), jnp.int32)
    pv = plsc.load_gather(tbl_v, [iota, zeros + NUM_SPATIAL])

    @plsc.parallel_loop(0, N_OUT, unroll=8)
    def _pr(j):
        plsc.store_scatter(pad_v, [zeros, iota, zeros + j], pv)

    @pl.when(wid < BATCH)
    def _send_pad_row():
        pltpu.async_copy(pad_v, out_hbm.at[wid, pl.ds(0, 1), :, :], sem2)

    def _gather_row(buf, r, spr):
        plsc.store_scatter(buf, [zeros + r, iota, zeros], pv)

        @plsc.parallel_loop(0, VREGS_PER_ROW, unroll=2)
        def _k(k):
            sv = plsc.load_gather(sp_v, [zeros + spr, iota + k * L])
            iv0 = (sv * jnp.float32(NUM_SPATIAL)).astype(jnp.int32)
            cvec = iota + (1 + k * L)
            for h in range(NUM_HEADS):
                vals = plsc.load_gather(tbl_v, [zeros + h, iv0])
                plsc.store_scatter(buf, [zeros + r, zeros + h, cvec], vals)

    bufs = (buf_a, buf_b, buf_c, buf_d)
    pending = {}
    for c in range(NCHUNK):
        buf = bufs[c % NRING]
        if c >= NRING:
            for cp in pending.pop(c - NRING):
                cp.wait()

        def _row(r, cc):
            _gather_row(buf, r, c * CROWS + r)
            return cc
        lax.fori_loop(0, CROWS, _row, 0)

        pending[c] = [
            pltpu.async_copy(
                buf,
                out_hbm.at[b, pl.ds(d0 + 1 + c * CROWS, CROWS), :, :], sem)
            for b in range(BATCH)
        ]
    for c in range(NCHUNK - NRING, NCHUNK):
        for cp in pending.pop(c):
            cp.wait()

    @pl.when(wid < BATCH)
    def _drain_pad_row():
        pltpu.make_async_copy(
            pad_v, out_hbm.at[wid, pl.ds(0, 1), :, :], sem2).wait()


@jax.jit
def _graph_attn_bias(spatial_pos, emb_weight):
    mesh = plsc.VectorSubcoreMesh(core_axis_name="c", subcore_axis_name="s")
    f = functools.partial(
        pl.kernel,
        mesh=mesh,
        out_type=jax.ShapeDtypeStruct((BATCH, N_OUT, NUM_HEADS, N_OUT),
                                      jnp.float32),
        scratch_types=[
            pltpu.VMEM((NUM_HEADS, NUM_SPATIAL + 1), jnp.float32),
            pltpu.VMEM((ROWS_PER_W, N_DATA), jnp.float32),
            pltpu.VMEM((1, NUM_HEADS, N_OUT), jnp.float32),
            pltpu.VMEM((CROWS, NUM_HEADS, N_OUT), jnp.float32),
            pltpu.VMEM((CROWS, NUM_HEADS, N_OUT), jnp.float32),
            pltpu.VMEM((CROWS, NUM_HEADS, N_OUT), jnp.float32),
            pltpu.VMEM((CROWS, NUM_HEADS, N_OUT), jnp.float32),
            pltpu.SemaphoreType.DMA,
            pltpu.SemaphoreType.DMA,
        ],
        compiler_params=pltpu.CompilerParams(needs_layout_passes=False),
    )(_sc_body)
    out_bihj = f(spatial_pos, emb_weight.T)
    return jnp.transpose(out_bihj, (0, 2, 1, 3))


def kernel(spatial_pos, x, emb_weight):
    del x
    return _graph_attn_bias(spatial_pos, emb_weight)
